# baseline (speedup 1.0000x reference)
import functools

import jax
import jax.numpy as jnp
from jax import lax
from jax.experimental import pallas as pl
from jax.experimental.pallas import tpu as pltpu
from jax.experimental.pallas import tpu_sc as plsc

BATCH = 16384
EMBED = 32
ROWS_PER_STREAM = 128 // EMBED

_info = plsc.get_sparse_core_info()
_NC, _NS = _info.num_cores, _info.num_subcores
_NW = _NC * _NS
_BPW = BATCH // _NW
_NBLK = _BPW // 16


def _build():
    mesh = plsc.VectorSubcoreMesh(core_axis_name="c", subcore_axis_name="s")
    out_t = jax.ShapeDtypeStruct((BATCH * EMBED,), jnp.float32)

    @functools.partial(
        pl.kernel,
        mesh=mesh,
        out_type=(out_t, out_t, out_t),
        compiler_params=pltpu.CompilerParams(needs_layout_passes=False),
        scratch_types=[
            pltpu.VMEM((_BPW,), jnp.int32),
            pltpu.VMEM((_BPW,), jnp.int32),
            pltpu.VMEM((_BPW, 128), jnp.float32),
            pltpu.VMEM((_BPW * EMBED,), jnp.float32),
            pltpu.SemaphoreType.DMA,
        ],
    )
    def gather3(users_hbm, pos_hbm, neg_hbm, utab_hbm, itab_hbm,
                out_u, out_p, out_n,
                idx_v, idx4_v, rows_v, out_flat, sem):
        wid = lax.axis_index("s") * _NC + lax.axis_index("c")
        base = wid * _BPW

        for idx_hbm, tab_hbm, out_hbm in (
                (users_hbm, utab_hbm, out_u),
                (pos_hbm, itab_hbm, out_p),
                (neg_hbm, itab_hbm, out_n)):
            pltpu.sync_copy(idx_hbm.at[pl.ds(base, _BPW)], idx_v)

            def shift(b, carry):
                s = pl.ds(pl.multiple_of(b * 16, 16), 16)
                idx4_v[s] = idx_v[s] >> 2
                return carry

            lax.fori_loop(0, _NBLK, shift, 0)
            pltpu.async_copy(tab_hbm.at[idx4_v], rows_v, sem).wait()

            iota16 = lax.iota(jnp.int32, 16)

            def row(i, carry):
                iv = plsc.load_gather(idx_v, [jnp.zeros((16,), jnp.int32) + i])
                lane0 = (iv & (ROWS_PER_STREAM - 1)) * EMBED + iota16
                r = rows_v.at[i]
                lo = plsc.load_gather(r, [lane0])
                hi = plsc.load_gather(r, [lane0 + 16])
                o = pl.multiple_of(i * EMBED, 16)
                o2 = pl.multiple_of(i * EMBED + 16, 16)
                out_flat[pl.ds(o, 16)] = lo
                out_flat[pl.ds(o2, 16)] = hi
                return carry

            lax.fori_loop(0, _BPW, row, 0)
            pltpu.sync_copy(
                out_flat, out_hbm.at[pl.ds(base * EMBED, _BPW * EMBED)])

    return gather3


_gather3 = _build()


def kernel(users, pos_items, neg_items, user_table, item_table):
    utab4 = user_table.reshape(-1, 128)
    itab4 = item_table.reshape(-1, 128)
    u, p, n = _gather3(users, pos_items, neg_items, utab4, itab4)
    return (u.reshape(BATCH, EMBED),
            p.reshape(BATCH, EMBED),
            n.reshape(BATCH, EMBED))


# speedup vs baseline: 1.5305x; 1.5305x over previous
import functools

import jax
import jax.numpy as jnp
from jax import lax
from jax.experimental import pallas as pl
from jax.experimental.pallas import tpu as pltpu
from jax.experimental.pallas import tpu_sc as plsc

BATCH = 16384
EMBED = 32

_info = plsc.get_sparse_core_info()
_NC, _NS = _info.num_cores, _info.num_subcores
_NW = _NC * _NS
_BPW = BATCH // _NW
_NBLK = _BPW // 16


def _build():
    mesh = plsc.VectorSubcoreMesh(core_axis_name="c", subcore_axis_name="s")
    out_t = jax.ShapeDtypeStruct((BATCH, EMBED), jnp.float32)

    @functools.partial(
        pl.kernel,
        mesh=mesh,
        out_type=(out_t, out_t, out_t),
        compiler_params=pltpu.CompilerParams(needs_layout_passes=False),
        scratch_types=[
            pltpu.VMEM((_BPW,), jnp.int32),
            pltpu.VMEM((_BPW, EMBED), jnp.float32),
            pltpu.SemaphoreType.DMA,
        ],
    )
    def gather3(users_hbm, pos_hbm, neg_hbm, utab_hbm, itab_hbm,
                out_u, out_p, out_n,
                idx_v, buf, sem):
        wid = lax.axis_index("s") * _NC + lax.axis_index("c")
        base = wid * _BPW
        iota16 = lax.iota(jnp.int32, 16)
        zeros16 = jnp.zeros((16,), jnp.int32)

        for idx_hbm, tab_hbm, out_hbm in (
                (users_hbm, utab_hbm, out_u),
                (pos_hbm, itab_hbm, out_p),
                (neg_hbm, itab_hbm, out_n)):
            pltpu.sync_copy(idx_hbm.at[pl.ds(base, _BPW)], idx_v)

            def block(b, carry):
                chunk = idx_v[pl.ds(pl.multiple_of(b * 16, 16), 16)]
                for l in range(16):
                    r = lax.reduce_sum_p.bind(
                        jnp.where(iota16 == l, chunk, zeros16), axes=(0,))
                    pltpu.async_copy(
                        tab_hbm.at[pl.ds(r, 1), :],
                        buf.at[pl.ds(b * 16 + l, 1), :],
                        sem)
                return carry

            lax.fori_loop(0, _NBLK, block, 0)
            pltpu.make_async_copy(
                tab_hbm.at[pl.ds(0, _BPW), :], buf, sem).wait()
            pltpu.sync_copy(buf, out_hbm.at[pl.ds(base, _BPW), :])

    return gather3


_gather3 = _build()


def kernel(users, pos_items, neg_items, user_table, item_table):
    return _gather3(users, pos_items, neg_items, user_table, item_table)
